# Initial kernel scaffold; baseline (speedup 1.0000x reference)
import jax
import jax.numpy as jnp
from jax.experimental import pallas as pl


def kernel(x_s, coord_s, x_l, coord_l, W_proj, b_proj, desc_feats, Wv, bv, Wu, bu, w_attn, b_attn):
    raise NotImplementedError("write your pallas kernel here")


# speedup vs baseline: 2.1442x; 2.1442x over previous
import jax
import jax.numpy as jnp
from jax.experimental import pallas as pl
from jax.experimental.pallas import tpu as pltpu

B, N, D = 2, 16384, 512
H = 256
C, K = 4, 8
EPS = 1e-12

BN = 2048
NBLK = N // BN
TOT = 2 * H + C * K


def _process(x, wp, bp, wall, ball, wattn):
    ss = jnp.sum(x * x, axis=1, keepdims=True)
    xn = x * jax.lax.rsqrt(jnp.maximum(ss, EPS * EPS))
    xpr = jnp.dot(xn, wp, preferred_element_type=jnp.float32) + bp
    rn2 = jax.lax.rsqrt(
        jnp.maximum(jnp.sum(xpr * xpr, axis=1, keepdims=True), EPS * EPS))
    z = jnp.dot(xpr, wall, preferred_element_type=jnp.float32) * rn2 + ball
    g = jnp.tanh(z[:, :H]) * jax.nn.sigmoid(z[:, H:2 * H])
    a = jnp.sum(g * wattn, axis=1, keepdims=True)
    sim = z[:, 2 * H:]
    e = jnp.exp(sim * (D ** -0.5))
    ki = jax.lax.broadcasted_iota(jnp.int32, (C * K, C), 0)
    ci = jax.lax.broadcasted_iota(jnp.int32, (C * K, C), 1)
    grp = (ki // K == ci).astype(jnp.float32)
    den = jnp.dot(e, grp, preferred_element_type=jnp.float32)
    num = jnp.dot(e * sim, grp, preferred_element_type=jnp.float32)
    return num / den, a, xpr, rn2


def _accum(a, xpr, rn2, acc_ref, m_ref, slot, blk):
    lm = jnp.max(a)
    m_old = m_ref[slot]
    m_new = jnp.where(blk == 0, lm, jnp.maximum(m_old, lm))
    m_ref[slot] = m_new
    w = jnp.exp(a - m_new) * rn2
    contrib = jnp.sum((xpr * w).reshape(BN // 8, 8, D), axis=0)

    @pl.when(blk == 0)
    def _():
        acc_ref[0] = contrib

    @pl.when(blk != 0)
    def _():
        acc_ref[0] = acc_ref[0] * jnp.exp(m_old - m_new) + contrib


def _fused_kernel(xs_ref, xl_ref, wp_ref, bp_ref, wall_ref, ball_ref,
                  wattn_ref, ss_ref, sl_ref, accs_ref, accl_ref, m_ref):
    blk = pl.program_id(1)
    wp = wp_ref[...]
    bp = bp_ref[...]
    wall = wall_ref[...]
    ball = ball_ref[...]
    wattn = wattn_ref[...]

    scores, a, xpr, rn2 = _process(xs_ref[0], wp, bp, wall, ball, wattn)
    ss_ref[0] = scores
    _accum(a, xpr, rn2, accs_ref, m_ref, 0, blk)

    scores, a, xpr, rn2 = _process(xl_ref[0], wp, bp, wall, ball, wattn)
    sl_ref[0] = scores
    _accum(a, xpr, rn2, accl_ref, m_ref, 1, blk)


def _l2n(x):
    return x / jnp.clip(jnp.linalg.norm(x, axis=-1, keepdims=True), EPS)


def kernel(x_s, coord_s, x_l, coord_l, W_proj, b_proj, desc_feats,
           Wv, bv, Wu, bu, w_attn, b_attn):
    desc2 = desc_feats.reshape(C * K, D)
    wall = jnp.concatenate([Wv, Wu, desc2.T], axis=1)
    ball = jnp.concatenate([bv, bu, jnp.zeros((C * K,), jnp.float32)])[None, :]
    bp = b_proj[None, :]
    wattn = w_attn[None, :]

    scores_s, scores_l, acc_s, acc_l = pl.pallas_call(
        _fused_kernel,
        grid=(B, NBLK),
        in_specs=[
            pl.BlockSpec((1, BN, D), lambda b, i: (b, i, 0)),
            pl.BlockSpec((1, BN, D), lambda b, i: (b, i, 0)),
            pl.BlockSpec((D, D), lambda b, i: (0, 0)),
            pl.BlockSpec((1, D), lambda b, i: (0, 0)),
            pl.BlockSpec((D, TOT), lambda b, i: (0, 0)),
            pl.BlockSpec((1, TOT), lambda b, i: (0, 0)),
            pl.BlockSpec((1, H), lambda b, i: (0, 0)),
        ],
        out_specs=[
            pl.BlockSpec((1, BN, C), lambda b, i: (b, i, 0)),
            pl.BlockSpec((1, BN, C), lambda b, i: (b, i, 0)),
            pl.BlockSpec((1, 8, D), lambda b, i: (b, 0, 0)),
            pl.BlockSpec((1, 8, D), lambda b, i: (b, 0, 0)),
        ],
        out_shape=[
            jax.ShapeDtypeStruct((B, N, C), jnp.float32),
            jax.ShapeDtypeStruct((B, N, C), jnp.float32),
            jax.ShapeDtypeStruct((B, 8, D), jnp.float32),
            jax.ShapeDtypeStruct((B, 8, D), jnp.float32),
        ],
        scratch_shapes=[pltpu.SMEM((2,), jnp.float32)],
        compiler_params=pltpu.CompilerParams(
            dimension_semantics=("arbitrary", "arbitrary"),
            vmem_limit_bytes=50 * 1024 * 1024,
        ),
        name="ver2f_fused",
    )(x_s, x_l, W_proj, bp, wall, ball, wattn)

    slide_s = _l2n(acc_s.sum(axis=1))
    slide_l = _l2n(acc_l.sum(axis=1))
    text = _l2n(jnp.max(desc_feats, axis=1))
    logits = slide_s @ text.T + slide_l @ text.T
    Y_prob = jax.nn.softmax(logits, axis=1)
    Y_hat = jnp.argmax(Y_prob, axis=1)
    return Y_prob, Y_hat, scores_s, scores_l
